# baseline (speedup 1.0000x reference)
import functools

import jax
import jax.numpy as jnp
from jax.experimental import pallas as pl
from jax.experimental.pallas import tpu as pltpu

B = 16384
D_IN = 17
D_OUT = 6
N_CLUSTERS = 4
H_EXP = 8
W_ROWS = 184


def _fused_kernel(xt_ref, w_ref, pred_ref, logits_ref):
    f32 = jnp.float32
    xT = xt_ref[...]

    a1t = w_ref[0:64, 0:D_IN]
    a2t = w_ref[64:96, 0:64]
    a3t = w_ref[96:100, 0:32]
    e1t = w_ref[104:136, 0:D_IN]
    e2 = w_ref[136:168, 0:32]
    e3t = w_ref[168:174, 0:32]
    gb1c = w_ref[176:177, 0:64].T
    gb2c = w_ref[177:178, 0:32].T
    gb3c = w_ref[178:179, 0:N_CLUSTERS].T
    b1c = w_ref[179:180, 0:32].T
    b2c = w_ref[180:181, 0:32].T
    eb3t = w_ref[176:182, 64:68]

    h = jnp.maximum(jnp.dot(a1t, xT, preferred_element_type=f32) + gb1c, 0.0)
    h = jnp.maximum(jnp.dot(a2t, h, preferred_element_type=f32) + gb2c, 0.0)
    logits = jnp.dot(a3t, h, preferred_element_type=f32) + gb3c
    logits_ref[...] = logits

    m = jnp.max(logits, axis=0, keepdims=True)
    iota4 = jax.lax.broadcasted_iota(jnp.int32, (N_CLUSTERS, B), 0)
    sel = jnp.min(jnp.where(logits == m, iota4, N_CLUSTERS),
                  axis=0, keepdims=True)

    h1 = jnp.maximum(jnp.dot(e1t, xT, preferred_element_type=f32) + b1c, 0.0)
    h2 = jnp.maximum(jnp.dot(e2, h1, preferred_element_type=f32) + b2c, 0.0)

    group = jax.lax.broadcasted_iota(jnp.int32, (N_CLUSTERS * H_EXP, B), 0) // H_EXP
    h2m = jnp.where(group == sel, h2, 0.0)
    onehot = (iota4 == sel).astype(f32)
    pred_ref[...] = (jnp.dot(e3t, h2m, preferred_element_type=f32)
                     + jnp.dot(eb3t, onehot, preferred_element_type=f32))


def _pack_weights(gW1, gb1, gW2, gb2, gW3, gb3, eW1, eb1, eW2, eb2, eW3, eb3):
    def pad(a, rows, cols=128):
        return jnp.pad(a, ((0, rows - a.shape[0]), (0, cols - a.shape[1])))

    e1t = eW1.transpose(0, 2, 1).reshape(N_CLUSTERS * H_EXP, D_IN)
    e2bd = jax.scipy.linalg.block_diag(*[eW2[e].T for e in range(N_CLUSTERS)])
    e3t = eW3.transpose(2, 0, 1).reshape(D_OUT, N_CLUSTERS * H_EXP)
    left = jnp.concatenate([
        pad(gb1.reshape(1, -1), 1, 64),
        pad(gb2.reshape(1, -1), 1, 64),
        pad(gb3.reshape(1, -1), 1, 64),
        pad(eb1.reshape(1, -1), 1, 64),
        pad(eb2.reshape(1, -1), 1, 64),
        jnp.zeros((3, 64), jnp.float32),
    ], axis=0)
    right = pad(eb3.T, 8, 64)
    bias_seg = jnp.concatenate([left, right], axis=1)
    return jnp.concatenate([
        pad(gW1.T, 64),
        pad(gW2.T, 32),
        pad(gW3.T, 8),
        pad(e1t, 32),
        pad(e2bd, 32),
        pad(e3t, 8),
        bias_seg,
    ], axis=0)


@functools.partial(jax.jit, static_argnames=())
def kernel(x, gW1, gb1, gW2, gb2, gW3, gb3, eW1, eb1, eW2, eb2, eW3, eb3):
    xt = x.T
    w = _pack_weights(gW1, gb1, gW2, gb2, gW3, gb3, eW1, eb1, eW2, eb2, eW3, eb3)

    predT, logitsT = pl.pallas_call(
        _fused_kernel,
        in_specs=[
            pl.BlockSpec((D_IN, B), lambda: (0, 0)),
            pl.BlockSpec((W_ROWS, 128), lambda: (0, 0)),
        ],
        out_specs=[
            pl.BlockSpec((D_OUT, B), lambda: (0, 0)),
            pl.BlockSpec((N_CLUSTERS, B), lambda: (0, 0)),
        ],
        out_shape=[
            jax.ShapeDtypeStruct((D_OUT, B), jnp.float32),
            jax.ShapeDtypeStruct((N_CLUSTERS, B), jnp.float32),
        ],
    )(xt, w)
    return predT.T, logitsT.T
